# Initial kernel scaffold; baseline (speedup 1.0000x reference)
import jax
import jax.numpy as jnp
from jax.experimental import pallas as pl


def kernel(interval, target, Wz, Wv):
    raise NotImplementedError("write your pallas kernel here")


# speedup vs baseline: 4.5242x; 4.5242x over previous
import functools

import jax
import jax.numpy as jnp
from jax import lax
from jax.experimental import pallas as pl
from jax.experimental.pallas import tpu as pltpu
from jax.experimental.pallas import tpu_sc as plsc

B = 16384
BAG = 50
K = 20
D = 64
NW = 32
BPW = B // NW
C = 16
NCHUNK = BPW // C
ZROWS = C * BAG
VROWS = C * K
SUB = 80
NZS = ZROWS // SUB
NVS = VROWS // SUB


def _sc_call(interval_flat, target_flat, wz, wv):
  mesh = plsc.VectorSubcoreMesh(core_axis_name="c", subcore_axis_name="s")

  @functools.partial(
      pl.kernel,
      out_type=jax.ShapeDtypeStruct((B * K,), jnp.float32),
      mesh=mesh,
      scratch_types=[
          pltpu.VMEM((ZROWS,), jnp.int32),
          pltpu.VMEM((VROWS,), jnp.int32),
          pltpu.VMEM((ZROWS, D), jnp.float32),
          pltpu.VMEM((VROWS, D), jnp.float32),
          pltpu.VMEM((VROWS,), jnp.float32),
          pltpu.VMEM((VROWS * 16,), jnp.float32),
          pltpu.SemaphoreType.DMA,
      ],
      compiler_params=pltpu.CompilerParams(
          needs_layout_passes=False, use_tc_tiling_on_sc=False),
  )
  def body(interval_hbm, target_hbm, wz_hbm, wv_hbm, out_hbm,
           iidx_v, tidx_v, zrows_v, vrows_v, y_v, psum_v, sem):
    wid = lax.axis_index("s") * 2 + lax.axis_index("c")
    bag0 = wid * BPW

    def chunk_body(ci, carry):
      cbag = bag0 + ci * C
      pltpu.sync_copy(interval_hbm.at[pl.ds(cbag * BAG, ZROWS)], iidx_v)
      pltpu.sync_copy(target_hbm.at[pl.ds(cbag * K, VROWS)], tidx_v)
      copies = []
      for j in range(NZS):
        copies.append(pltpu.async_copy(
            wz_hbm.at[iidx_v.at[pl.ds(j * SUB, SUB)]],
            zrows_v.at[pl.ds(j * SUB, SUB)], sem))
      for j in range(NVS):
        copies.append(pltpu.async_copy(
            wv_hbm.at[tidx_v.at[pl.ds(j * SUB, SUB)]],
            vrows_v.at[pl.ds(j * SUB, SUB)], sem))
      for cp in copies:
        cp.wait()

      def bag_body(b, carry2):
        zbase = b * BAG

        def row_body(r, acc):
          a0, a1, a2, a3 = acc
          row = zbase + r
          a0 = a0 + zrows_v[row, pl.ds(0, 16)]
          a1 = a1 + zrows_v[row, pl.ds(16, 16)]
          a2 = a2 + zrows_v[row, pl.ds(32, 16)]
          a3 = a3 + zrows_v[row, pl.ds(48, 16)]
          return (a0, a1, a2, a3)

        zero = jnp.zeros((16,), jnp.float32)
        s0, s1, s2, s3 = lax.fori_loop(0, BAG, row_body,
                                       (zero, zero, zero, zero))
        scale = jnp.float32(1.0 / BAG)
        z0, z1, z2, z3 = s0 * scale, s1 * scale, s2 * scale, s3 * scale

        vbase = b * K
        for k in range(K):
          row = vbase + k
          p = z0 * vrows_v[row, pl.ds(0, 16)]
          p = p + z1 * vrows_v[row, pl.ds(16, 16)]
          p = p + z2 * vrows_v[row, pl.ds(32, 16)]
          p = p + z3 * vrows_v[row, pl.ds(48, 16)]
          psum_v[pl.ds(row * 16, 16)] = p
        return carry2

      lax.fori_loop(0, C, bag_body, 0)

      lanes = lax.iota(jnp.int32, 16)

      def red_body(g, carry2):
        base = (g * 16 + lanes) * 16

        def d_body(d, acc):
          return acc + plsc.load_gather(psum_v, [base + d])

        acc = lax.fori_loop(0, 16, d_body, jnp.zeros((16,), jnp.float32))
        y_v[pl.ds(g * 16, 16)] = acc
        return carry2

      lax.fori_loop(0, VROWS // 16, red_body, 0)
      pltpu.sync_copy(y_v, out_hbm.at[pl.ds(cbag * K, VROWS)])
      return carry

    lax.fori_loop(0, NCHUNK, chunk_body, 0)

  return body(interval_flat, target_flat, wz, wv)


def kernel(interval, target, Wz, Wv):
  return _sc_call(interval.reshape(-1), target.reshape(-1), Wz, Wv)
